# speedup vs baseline: 10.1724x; 10.1724x over previous
import functools

import jax
import jax.numpy as jnp
from jax import lax
from jax.experimental import pallas as pl
from jax.experimental.pallas import tpu as pltpu
from jax.experimental.pallas import tpu_sc as plsc

NC = 2
NS = 16
NW = NC * NS


def _sc_aggregate(x, row, col, z2, z1, n_pad, ch, nch):
    n, d = x.shape
    rows_per_tile = n_pad // NS
    epw = ch * nch

    mesh = plsc.VectorSubcoreMesh(core_axis_name="c", subcore_axis_name="s")

    def body(x_hbm, row_hbm, col_hbm, z2_hbm, z1_hbm, agg_out, deg_out,
             colv, rowv, rows, ones, agg_sh, deg_sh, sem):
        c = lax.axis_index("c")
        s = lax.axis_index("s")
        wid = s * NC + c

        tile_lo = s * rows_per_tile
        pltpu.sync_copy(z2_hbm.at[pl.ds(tile_lo, rows_per_tile)],
                        agg_sh.at[pl.ds(tile_lo, rows_per_tile)])
        pltpu.sync_copy(z1_hbm.at[pl.ds(tile_lo, rows_per_tile)],
                        deg_sh.at[pl.ds(tile_lo, rows_per_tile)])
        for i in range(ch // 16):
            ones[pl.ds(i * 16, 16)] = jnp.ones((16,), jnp.float32)
        plsc.subcore_barrier()

        base = wid * epw

        def chunk(j, carry):
            off = base + j * ch
            pltpu.sync_copy(col_hbm.at[pl.ds(off, ch)], colv)
            pltpu.sync_copy(row_hbm.at[pl.ds(off, ch)], rowv)
            pltpu.async_copy(x_hbm.at[colv], rows, sem).wait()
            pltpu.sync_copy(rows, agg_sh.at[rowv], add=True)
            pltpu.sync_copy(ones, deg_sh.at[rowv], add=True)
            return carry

        lax.fori_loop(0, nch, chunk, 0)
        plsc.subcore_barrier()

        pltpu.sync_copy(agg_sh.at[pl.ds(tile_lo, rows_per_tile)],
                        agg_out.at[c, pl.ds(tile_lo, rows_per_tile)])
        pltpu.sync_copy(deg_sh.at[pl.ds(tile_lo, rows_per_tile)],
                        deg_out.at[c, pl.ds(tile_lo, rows_per_tile)])

    fn = pl.kernel(
        body,
        out_type=(jax.ShapeDtypeStruct((NC, n_pad, d), jnp.float32),
                  jax.ShapeDtypeStruct((NC, n_pad), jnp.float32)),
        mesh=mesh,
        scratch_types=[
            pltpu.VMEM((ch,), jnp.int32),
            pltpu.VMEM((ch,), jnp.int32),
            pltpu.VMEM((ch, d), jnp.float32),
            pltpu.VMEM((ch,), jnp.float32),
            pltpu.VMEM_SHARED((n_pad, d), jnp.float32),
            pltpu.VMEM_SHARED((n_pad,), jnp.float32),
            pltpu.SemaphoreType.DMA,
        ],
    )
    return fn(x, row, col, z2, z1)


def _tc_dense_body(agg_ref, deg_ref, Word_ref, bord_ref, Wg_ref, bg_ref,
                   Worig_ref, borig_ref, W1a_ref, W1b_ref, b1_ref,
                   W2_ref, b2_ref, out_ref):
    a = agg_ref[0] + agg_ref[1]
    dg = deg_ref[0] + deg_ref[1]

    es = []
    ss = []
    for k in range(3):
        e = jnp.maximum(
            jnp.dot(a, Word_ref[k], preferred_element_type=jnp.float32)
            + dg * bord_ref[k], 0.0)
        s = (jnp.dot(e, Wg_ref[k], preferred_element_type=jnp.float32)
             + bg_ref[k])
        es.append(e)
        ss.append(s)

    m = jnp.maximum(jnp.maximum(ss[0], ss[1]), ss[2])
    p = [jnp.exp(s - m) for s in ss]
    tot = p[0] + p[1] + p[2]
    h_high = (p[0] * es[0] + p[1] * es[1] + p[2] * es[2]) / tot

    h_orig = jnp.maximum(
        jnp.dot(a, Worig_ref[...], preferred_element_type=jnp.float32)
        + dg * borig_ref[...], 0.0)

    z1 = jnp.maximum(
        jnp.dot(h_orig, W1a_ref[...], preferred_element_type=jnp.float32)
        + jnp.dot(h_high, W1b_ref[...], preferred_element_type=jnp.float32)
        + b1_ref[...], 0.0)
    out_ref[...] = (jnp.dot(z1, W2_ref[...], preferred_element_type=jnp.float32)
                    + b2_ref[...])


def _tc_dense(agg2, deg2, W_ord, b_ord, W_gate, b_gate, W_orig, b_orig,
              W1a, W1b, b_c1, W_c2, b_c2, n_pad, blk):
    d = agg2.shape[-1]
    d_out = W_c2.shape[-1]
    grid = n_pad // blk

    def full(shape):
        return pl.BlockSpec(shape, lambda i, _s=shape: tuple(0 for _ in _s))

    return pl.pallas_call(
        _tc_dense_body,
        grid=(grid,),
        in_specs=[
            pl.BlockSpec((NC, blk, d), lambda i: (0, i, 0)),
            pl.BlockSpec((NC, blk, 1), lambda i: (0, i, 0)),
            full(W_ord.shape),
            full(b_ord.shape),
            full(W_gate.shape),
            full(b_gate.shape),
            full(W_orig.shape),
            full(b_orig.shape),
            full(W1a.shape),
            full(W1b.shape),
            full(b_c1.shape),
            full(W_c2.shape),
            full(b_c2.shape),
        ],
        out_specs=pl.BlockSpec((blk, d_out), lambda i: (i, 0)),
        out_shape=jax.ShapeDtypeStruct((n_pad, d_out), jnp.float32),
    )(agg2, deg2, W_ord, b_ord, W_gate, b_gate, W_orig, b_orig,
      W1a, W1b, b_c1, W_c2, b_c2)


def kernel(x, edge_index, W_ord, b_ord, W_gate, b_gate, W_orig, b_orig,
           W_c1, b_c1, W_c2, b_c2, gamma):
    n, d = x.shape
    e = edge_index.shape[1]
    n_pad = ((n + 1023) // 1024) * 1024

    epw = e // NW
    ch = next(c for c in (80, 64, 40, 32, 16, 8) if epw % c == 0)
    nch = epw // ch

    row = edge_index[0]
    col = edge_index[1]
    z2 = jnp.zeros((n_pad, d), jnp.float32)
    z1 = jnp.zeros((n_pad,), jnp.float32)

    agg2, deg2 = _sc_aggregate(x, row, col, z2, z1, n_pad, ch, nch)

    W1a = W_c1[:d]
    W1b = gamma * W_c1[d:]
    out = _tc_dense(
        agg2, deg2[:, :, None],
        W_ord, b_ord[:, None, :], W_gate, b_gate[:, None, :],
        W_orig, b_orig[None, :], W1a, W1b, b_c1[None, :],
        W_c2, b_c2[None, :], n_pad, blk=1280)
    return out[:n]


# speedup vs baseline: 15.1112x; 1.4855x over previous
import functools

import jax
import jax.numpy as jnp
from jax import lax
from jax.experimental import pallas as pl
from jax.experimental.pallas import tpu as pltpu
from jax.experimental.pallas import tpu_sc as plsc

NC = 2
NS = 16
NW = NC * NS


R_GRP = 5


def _sc_aggregate(x, row, col, z2, z1, n_pad, ch, ngrp):
    n, d = x.shape
    rows_per_tile = n_pad // NS
    epg = ch * R_GRP

    mesh = plsc.VectorSubcoreMesh(core_axis_name="c", subcore_axis_name="s")

    def body(x_hbm, row_hbm, col_hbm, z2_hbm, z1_hbm, agg_out, deg_out,
             ones, agg_sh, deg_sh, *bufs):
        rows = bufs[0:R_GRP]
        rowv = bufs[R_GRP:2 * R_GRP]
        colv = bufs[2 * R_GRP:3 * R_GRP]
        gsem = bufs[3 * R_GRP:4 * R_GRP]
        isem = bufs[4 * R_GRP:5 * R_GRP]
        c = lax.axis_index("c")
        s = lax.axis_index("s")
        wid = s * NC + c

        tile_lo = s * rows_per_tile
        pltpu.sync_copy(z2_hbm.at[pl.ds(tile_lo, rows_per_tile)],
                        agg_sh.at[pl.ds(tile_lo, rows_per_tile)])
        pltpu.sync_copy(z1_hbm.at[pl.ds(tile_lo, rows_per_tile)],
                        deg_sh.at[pl.ds(tile_lo, rows_per_tile)])
        for off in sorted({min(i * 16, ch - 16)
                           for i in range((ch + 15) // 16)}):
            ones[pl.ds(off, 16)] = jnp.ones((16,), jnp.float32)
        plsc.subcore_barrier()

        base = wid * (epg * ngrp)

        def group(i, carry):
            goff = base + i * epg
            idescs = []
            for u in range(R_GRP):
                off = goff + u * ch
                idescs.append(
                    (pltpu.async_copy(col_hbm.at[pl.ds(off, ch)], colv[u],
                                      isem[u]),
                     pltpu.async_copy(row_hbm.at[pl.ds(off, ch)], rowv[u],
                                      isem[u])))
            gdescs = []
            for u in range(R_GRP):
                idescs[u][0].wait()
                idescs[u][1].wait()
                gdescs.append(
                    pltpu.async_copy(x_hbm.at[colv[u]], rows[u], gsem[u]))
            for u in range(R_GRP):
                gdescs[u].wait()
                pltpu.sync_copy(rows[u], agg_sh.at[rowv[u]], add=True)
                pltpu.sync_copy(ones, deg_sh.at[rowv[u]], add=True)
            return carry

        lax.fori_loop(0, ngrp, group, 0)
        plsc.subcore_barrier()

        pltpu.sync_copy(agg_sh.at[pl.ds(tile_lo, rows_per_tile)],
                        agg_out.at[c, pl.ds(tile_lo, rows_per_tile)])
        pltpu.sync_copy(deg_sh.at[pl.ds(tile_lo, rows_per_tile)],
                        deg_out.at[c, pl.ds(tile_lo, rows_per_tile)])

    fn = pl.kernel(
        body,
        out_type=(jax.ShapeDtypeStruct((NC, n_pad, d), jnp.float32),
                  jax.ShapeDtypeStruct((NC, n_pad), jnp.float32)),
        mesh=mesh,
        scratch_types=[
            pltpu.VMEM((ch,), jnp.float32),
            pltpu.VMEM_SHARED((n_pad, d), jnp.float32),
            pltpu.VMEM_SHARED((n_pad,), jnp.float32),
        ] + [pltpu.VMEM((ch, d), jnp.float32) for _ in range(R_GRP)]
          + [pltpu.VMEM((ch,), jnp.int32) for _ in range(2 * R_GRP)]
          + [pltpu.SemaphoreType.DMA for _ in range(2 * R_GRP)],
    )
    return fn(x, row, col, z2, z1)


def _tc_dense_body(agg_ref, deg_ref, Word_ref, bord_ref, Wg_ref, bg_ref,
                   Worig_ref, borig_ref, W1a_ref, W1b_ref, b1_ref,
                   W2_ref, b2_ref, out_ref):
    a = agg_ref[0] + agg_ref[1]
    dg = deg_ref[0] + deg_ref[1]

    es = []
    ss = []
    for k in range(3):
        e = jnp.maximum(
            jnp.dot(a, Word_ref[k], preferred_element_type=jnp.float32)
            + dg * bord_ref[k], 0.0)
        s = (jnp.dot(e, Wg_ref[k], preferred_element_type=jnp.float32)
             + bg_ref[k])
        es.append(e)
        ss.append(s)

    m = jnp.maximum(jnp.maximum(ss[0], ss[1]), ss[2])
    p = [jnp.exp(s - m) for s in ss]
    tot = p[0] + p[1] + p[2]
    h_high = (p[0] * es[0] + p[1] * es[1] + p[2] * es[2]) / tot

    h_orig = jnp.maximum(
        jnp.dot(a, Worig_ref[...], preferred_element_type=jnp.float32)
        + dg * borig_ref[...], 0.0)

    z1 = jnp.maximum(
        jnp.dot(h_orig, W1a_ref[...], preferred_element_type=jnp.float32)
        + jnp.dot(h_high, W1b_ref[...], preferred_element_type=jnp.float32)
        + b1_ref[...], 0.0)
    out_ref[...] = (jnp.dot(z1, W2_ref[...], preferred_element_type=jnp.float32)
                    + b2_ref[...])


def _tc_dense(agg2, deg2, W_ord, b_ord, W_gate, b_gate, W_orig, b_orig,
              W1a, W1b, b_c1, W_c2, b_c2, n_pad, blk):
    d = agg2.shape[-1]
    d_out = W_c2.shape[-1]
    grid = n_pad // blk

    def full(shape):
        return pl.BlockSpec(shape, lambda i, _s=shape: tuple(0 for _ in _s))

    return pl.pallas_call(
        _tc_dense_body,
        grid=(grid,),
        in_specs=[
            pl.BlockSpec((NC, blk, d), lambda i: (0, i, 0)),
            pl.BlockSpec((NC, blk, 1), lambda i: (0, i, 0)),
            full(W_ord.shape),
            full(b_ord.shape),
            full(W_gate.shape),
            full(b_gate.shape),
            full(W_orig.shape),
            full(b_orig.shape),
            full(W1a.shape),
            full(W1b.shape),
            full(b_c1.shape),
            full(W_c2.shape),
            full(b_c2.shape),
        ],
        out_specs=pl.BlockSpec((blk, d_out), lambda i: (i, 0)),
        out_shape=jax.ShapeDtypeStruct((n_pad, d_out), jnp.float32),
    )(agg2, deg2, W_ord, b_ord, W_gate, b_gate, W_orig, b_orig,
      W1a, W1b, b_c1, W_c2, b_c2)


def kernel(x, edge_index, W_ord, b_ord, W_gate, b_gate, W_orig, b_orig,
           W_c1, b_c1, W_c2, b_c2, gamma):
    n, d = x.shape
    e = edge_index.shape[1]
    n_pad = ((n + 1023) // 1024) * 1024

    epw = e // NW
    ch = next(c for c in (40, 32, 16, 8)
              if epw % (c * R_GRP) == 0)
    ngrp = epw // (ch * R_GRP)

    row = edge_index[0]
    col = edge_index[1]
    z2 = jnp.zeros((n_pad, d), jnp.float32)
    z1 = jnp.zeros((n_pad,), jnp.float32)

    agg2, deg2 = _sc_aggregate(x, row, col, z2, z1, n_pad, ch, ngrp)

    W1a = W_c1[:d]
    W1b = gamma * W_c1[d:]
    out = _tc_dense(
        agg2, deg2[:, :, None],
        W_ord, b_ord[:, None, :], W_gate, b_gate[:, None, :],
        W_orig, b_orig[None, :], W1a, W1b, b_c1[None, :],
        W_c2, b_c2[None, :], n_pad, blk=n_pad // 8)
    return out[:n]


# speedup vs baseline: 17.4141x; 1.1524x over previous
import jax
import jax.numpy as jnp
from jax import lax
from jax.experimental import pallas as pl
from jax.experimental.pallas import tpu as pltpu
from jax.experimental.pallas import tpu_sc as plsc

NC = 2
NS = 16
NW = NC * NS
R_GRP = 5


def _sc_aggregate(x, idx3, z2, z1, n_pad, ch, ngrp):
    n, d = x.shape
    rows_per_tile = n_pad // NS

    mesh = plsc.VectorSubcoreMesh(core_axis_name="c", subcore_axis_name="s")

    def body(x_hbm, idx_hbm, z2_hbm, z1_hbm, agg_out, deg_out,
             idxA, idxB, ones, agg_sh, deg_sh, *bufs):
        rows = bufs[0:R_GRP]
        gsem = bufs[R_GRP:2 * R_GRP]
        isemA = bufs[2 * R_GRP]
        isemB = bufs[2 * R_GRP + 1]
        c = lax.axis_index("c")
        s = lax.axis_index("s")
        wid = s * NC + c

        tile_lo = s * rows_per_tile
        pltpu.sync_copy(z2_hbm, agg_sh.at[pl.ds(tile_lo, rows_per_tile)])
        pltpu.sync_copy(z1_hbm, deg_sh.at[pl.ds(tile_lo, rows_per_tile)])
        for off in sorted({min(i * 16, ch - 16)
                           for i in range((ch + 15) // 16)}):
            ones[pl.ds(off, 16)] = jnp.ones((16,), jnp.float32)
        plsc.subcore_barrier()

        gbase = wid * ngrp

        def fire(idx, u):
            return pltpu.async_copy(x_hbm.at[idx.at[2 * u + 1]], rows[u],
                                    gsem[u])

        def drain_refill(idx_cur, idx_nxt, gds, refill):
            nds = []
            for u in range(R_GRP):
                gds[u].wait()
                pltpu.sync_copy(rows[u], agg_sh.at[idx_cur.at[2 * u]],
                                add=True)
                if refill:
                    nds.append(fire(idx_nxt, u))
                pltpu.sync_copy(ones, deg_sh.at[idx_cur.at[2 * u]], add=True)
            return nds

        pltpu.sync_copy(idx_hbm.at[gbase], idxA)
        g_in_flight = [fire(idxA, u) for u in range(R_GRP)]
        pltpu.async_copy(idx_hbm.at[gbase + 1], idxB, isemB)

        def pair(i, carry):
            pltpu.make_async_copy(idx_hbm.at[gbase], idxB, isemB).wait()
            gB = drain_refill(idxA, idxB, g_in_flight, True)
            pltpu.async_copy(idx_hbm.at[gbase + 2 * i + 2], idxA, isemA)
            pltpu.make_async_copy(idx_hbm.at[gbase], idxA, isemA).wait()
            gA = drain_refill(idxB, idxA, gB, True)
            for u in range(R_GRP):
                g_in_flight[u] = gA[u]
            pltpu.async_copy(idx_hbm.at[gbase + 2 * i + 3], idxB, isemB)
            return carry

        lax.fori_loop(0, ngrp // 2 - 1, pair, 0)

        pltpu.make_async_copy(idx_hbm.at[gbase], idxB, isemB).wait()
        gB = drain_refill(idxA, idxB, g_in_flight, True)
        drain_refill(idxB, idxB, gB, False)

        plsc.subcore_barrier()

        pltpu.sync_copy(agg_sh.at[pl.ds(tile_lo, rows_per_tile)],
                        agg_out.at[c, pl.ds(tile_lo, rows_per_tile)])
        pltpu.sync_copy(deg_sh.at[pl.ds(tile_lo, rows_per_tile)],
                        deg_out.at[c, pl.ds(tile_lo, rows_per_tile)])

    fn = pl.kernel(
        body,
        out_type=(jax.ShapeDtypeStruct((NC, n_pad, d), jnp.float32),
                  jax.ShapeDtypeStruct((NC, n_pad), jnp.float32)),
        mesh=mesh,
        scratch_types=[
            pltpu.VMEM((2 * R_GRP, ch), jnp.int32),
            pltpu.VMEM((2 * R_GRP, ch), jnp.int32),
            pltpu.VMEM((ch,), jnp.float32),
            pltpu.VMEM_SHARED((n_pad, d), jnp.float32),
            pltpu.VMEM_SHARED((n_pad,), jnp.float32),
        ] + [pltpu.VMEM((ch, d), jnp.float32) for _ in range(R_GRP)]
          + [pltpu.SemaphoreType.DMA for _ in range(R_GRP + 2)],
    )
    return fn(x, idx3, z2, z1)


def _tc_dense_body(agg_ref, deg_ref, Word_ref, bord_ref, Wg_ref, bg_ref,
                   Worig_ref, borig_ref, W1a_ref, W1b_ref, b1_ref,
                   W2_ref, b2_ref, out_ref):
    a = agg_ref[0] + agg_ref[1]
    dg = deg_ref[0] + deg_ref[1]

    es = []
    ss = []
    for k in range(3):
        e = jnp.maximum(
            jnp.dot(a, Word_ref[k], preferred_element_type=jnp.float32)
            + dg * bord_ref[k], 0.0)
        s = (jnp.dot(e, Wg_ref[k], preferred_element_type=jnp.float32)
             + bg_ref[k])
        es.append(e)
        ss.append(s)

    m = jnp.maximum(jnp.maximum(ss[0], ss[1]), ss[2])
    p = [jnp.exp(s - m) for s in ss]
    tot = p[0] + p[1] + p[2]
    h_high = (p[0] * es[0] + p[1] * es[1] + p[2] * es[2]) / tot

    h_orig = jnp.maximum(
        jnp.dot(a, Worig_ref[...], preferred_element_type=jnp.float32)
        + dg * borig_ref[...], 0.0)

    z1 = jnp.maximum(
        jnp.dot(h_orig, W1a_ref[...], preferred_element_type=jnp.float32)
        + jnp.dot(h_high, W1b_ref[...], preferred_element_type=jnp.float32)
        + b1_ref[...], 0.0)
    out_ref[...] = (jnp.dot(z1, W2_ref[...], preferred_element_type=jnp.float32)
                    + b2_ref[...])


def _tc_dense(agg2, deg2, W_ord, b_ord, W_gate, b_gate, W_orig, b_orig,
              W1a, W1b, b_c1, W_c2, b_c2, n_pad, blk):
    d = agg2.shape[-1]
    d_out = W_c2.shape[-1]
    grid = n_pad // blk

    def full(shape):
        return pl.BlockSpec(shape, lambda i, _s=shape: tuple(0 for _ in _s))

    return pl.pallas_call(
        _tc_dense_body,
        grid=(grid,),
        in_specs=[
            pl.BlockSpec((NC, blk, d), lambda i: (0, i, 0)),
            pl.BlockSpec((NC, blk, 1), lambda i: (0, i, 0)),
            full(W_ord.shape),
            full(b_ord.shape),
            full(W_gate.shape),
            full(b_gate.shape),
            full(W_orig.shape),
            full(b_orig.shape),
            full(W1a.shape),
            full(W1b.shape),
            full(b_c1.shape),
            full(W_c2.shape),
            full(b_c2.shape),
        ],
        out_specs=pl.BlockSpec((blk, d_out), lambda i: (i, 0)),
        out_shape=jax.ShapeDtypeStruct((n_pad, d_out), jnp.float32),
    )(agg2, deg2, W_ord, b_ord, W_gate, b_gate, W_orig, b_orig,
      W1a, W1b, b_c1, W_c2, b_c2)


def kernel(x, edge_index, W_ord, b_ord, W_gate, b_gate, W_orig, b_orig,
           W_c1, b_c1, W_c2, b_c2, gamma):
    n, d = x.shape
    e = edge_index.shape[1]
    n_pad = ((n + 1023) // 1024) * 1024

    epw = e // NW
    ch = next(c for c in (40, 32, 16, 8)
              if epw % (c * R_GRP) == 0 and (epw // (c * R_GRP)) % 2 == 0)
    ngrp = epw // (ch * R_GRP)

    idx3 = (edge_index.reshape(2, NW, ngrp, R_GRP, ch)
            .transpose(1, 2, 3, 0, 4)
            .reshape(NW * ngrp, 2 * R_GRP, ch))
    z2 = jnp.zeros((n_pad // NS, d), jnp.float32)
    z1 = jnp.zeros((n_pad // NS,), jnp.float32)

    agg2, deg2 = _sc_aggregate(x, idx3, z2, z1, n_pad, ch, ngrp)

    W1a = W_c1[:d]
    W1b = gamma * W_c1[d:]
    out = _tc_dense(
        agg2, deg2[:, :, None],
        W_ord, b_ord[:, None, :], W_gate, b_gate[:, None, :],
        W_orig, b_orig[None, :], W1a, W1b, b_c1[None, :],
        W_c2, b_c2[None, :], n_pad, blk=n_pad // 8)
    return out[:n]


# speedup vs baseline: 17.8886x; 1.0272x over previous
import jax
import jax.numpy as jnp
from jax import lax
from jax.experimental import pallas as pl
from jax.experimental.pallas import tpu as pltpu
from jax.experimental.pallas import tpu_sc as plsc

NC = 2
NS = 16
NW = NC * NS
R_GRP = 5


def _sc_aggregate(x, idx3, z2, z1, n_pad, ch, ngrp):
    n, d = x.shape
    rows_per_tile = n_pad // NS

    mesh = plsc.VectorSubcoreMesh(core_axis_name="c", subcore_axis_name="s")

    def body(x_hbm, idx_hbm, z2_hbm, z1_hbm, agg_out, deg_out,
             idxA, idxB, ones, agg_sh, deg_sh, *bufs):
        rows = bufs[0:R_GRP]
        gsem = bufs[R_GRP:2 * R_GRP]
        ssem = bufs[2 * R_GRP:3 * R_GRP]
        isemA = bufs[3 * R_GRP]
        isemB = bufs[3 * R_GRP + 1]
        dsem = bufs[3 * R_GRP + 2]
        c = lax.axis_index("c")
        s = lax.axis_index("s")
        wid = s * NC + c

        tile_lo = s * rows_per_tile
        pltpu.sync_copy(z2_hbm, agg_sh.at[pl.ds(tile_lo, rows_per_tile)])
        pltpu.sync_copy(z1_hbm, deg_sh.at[pl.ds(tile_lo, rows_per_tile)])
        for off in sorted({min(i * 16, ch - 16)
                           for i in range((ch + 15) // 16)}):
            ones[pl.ds(off, 16)] = jnp.ones((16,), jnp.float32)
        plsc.subcore_barrier()

        gbase = wid * ngrp

        def fire(idx, u):
            return pltpu.async_copy(x_hbm.at[idx.at[2 * u + 1]], rows[u],
                                    gsem[u])

        def drain_refill(idx_cur, idx_nxt, gds, refill):
            sds = []
            dds = []
            for u in range(R_GRP):
                gds[u].wait()
                sds.append(pltpu.async_copy(
                    rows[u], agg_sh.at[idx_cur.at[2 * u]], ssem[u],
                    add=True))
                dds.append(pltpu.async_copy(
                    ones, deg_sh.at[idx_cur.at[2 * u]], dsem, add=True))
            nds = []
            for u in range(R_GRP):
                sds[u].wait()
                if refill:
                    nds.append(fire(idx_nxt, u))
            for u in range(R_GRP):
                dds[u].wait()
            return nds

        pltpu.sync_copy(idx_hbm.at[gbase], idxA)
        g_in_flight = [fire(idxA, u) for u in range(R_GRP)]
        pltpu.async_copy(idx_hbm.at[gbase + 1], idxB, isemB)

        def pair(i, carry):
            pltpu.make_async_copy(idx_hbm.at[gbase], idxB, isemB).wait()
            gB = drain_refill(idxA, idxB, g_in_flight, True)
            pltpu.async_copy(idx_hbm.at[gbase + 2 * i + 2], idxA, isemA)
            pltpu.make_async_copy(idx_hbm.at[gbase], idxA, isemA).wait()
            gA = drain_refill(idxB, idxA, gB, True)
            for u in range(R_GRP):
                g_in_flight[u] = gA[u]
            pltpu.async_copy(idx_hbm.at[gbase + 2 * i + 3], idxB, isemB)
            return carry

        lax.fori_loop(0, ngrp // 2 - 1, pair, 0)

        pltpu.make_async_copy(idx_hbm.at[gbase], idxB, isemB).wait()
        gB = drain_refill(idxA, idxB, g_in_flight, True)
        drain_refill(idxB, idxB, gB, False)

        plsc.subcore_barrier()

        pltpu.sync_copy(agg_sh.at[pl.ds(tile_lo, rows_per_tile)],
                        agg_out.at[c, pl.ds(tile_lo, rows_per_tile)])
        pltpu.sync_copy(deg_sh.at[pl.ds(tile_lo, rows_per_tile)],
                        deg_out.at[c, pl.ds(tile_lo, rows_per_tile)])

    fn = pl.kernel(
        body,
        out_type=(jax.ShapeDtypeStruct((NC, n_pad, d), jnp.float32),
                  jax.ShapeDtypeStruct((NC, n_pad), jnp.float32)),
        mesh=mesh,
        scratch_types=[
            pltpu.VMEM((2 * R_GRP, ch), jnp.int32),
            pltpu.VMEM((2 * R_GRP, ch), jnp.int32),
            pltpu.VMEM((ch,), jnp.float32),
            pltpu.VMEM_SHARED((n_pad, d), jnp.float32),
            pltpu.VMEM_SHARED((n_pad,), jnp.float32),
        ] + [pltpu.VMEM((ch, d), jnp.float32) for _ in range(R_GRP)]
          + [pltpu.SemaphoreType.DMA for _ in range(2 * R_GRP + 3)],
    )
    return fn(x, idx3, z2, z1)


def _tc_dense_body(agg_ref, deg_ref, Word_ref, bord_ref, Wg_ref, bg_ref,
                   Worig_ref, borig_ref, W1a_ref, W1b_ref, b1_ref,
                   W2_ref, b2_ref, out_ref):
    a = agg_ref[0] + agg_ref[1]
    dg = deg_ref[0] + deg_ref[1]

    es = []
    ss = []
    for k in range(3):
        e = jnp.maximum(
            jnp.dot(a, Word_ref[k], preferred_element_type=jnp.float32)
            + dg * bord_ref[k], 0.0)
        s = (jnp.dot(e, Wg_ref[k], preferred_element_type=jnp.float32)
             + bg_ref[k])
        es.append(e)
        ss.append(s)

    m = jnp.maximum(jnp.maximum(ss[0], ss[1]), ss[2])
    p = [jnp.exp(s - m) for s in ss]
    tot = p[0] + p[1] + p[2]
    h_high = (p[0] * es[0] + p[1] * es[1] + p[2] * es[2]) / tot

    h_orig = jnp.maximum(
        jnp.dot(a, Worig_ref[...], preferred_element_type=jnp.float32)
        + dg * borig_ref[...], 0.0)

    z1 = jnp.maximum(
        jnp.dot(h_orig, W1a_ref[...], preferred_element_type=jnp.float32)
        + jnp.dot(h_high, W1b_ref[...], preferred_element_type=jnp.float32)
        + b1_ref[...], 0.0)
    out_ref[...] = (jnp.dot(z1, W2_ref[...], preferred_element_type=jnp.float32)
                    + b2_ref[...])


def _tc_dense(agg2, deg2, W_ord, b_ord, W_gate, b_gate, W_orig, b_orig,
              W1a, W1b, b_c1, W_c2, b_c2, n_pad, blk):
    d = agg2.shape[-1]
    d_out = W_c2.shape[-1]
    grid = n_pad // blk

    def full(shape):
        return pl.BlockSpec(shape, lambda i, _s=shape: tuple(0 for _ in _s))

    return pl.pallas_call(
        _tc_dense_body,
        grid=(grid,),
        in_specs=[
            pl.BlockSpec((NC, blk, d), lambda i: (0, i, 0)),
            pl.BlockSpec((NC, blk, 1), lambda i: (0, i, 0)),
            full(W_ord.shape),
            full(b_ord.shape),
            full(W_gate.shape),
            full(b_gate.shape),
            full(W_orig.shape),
            full(b_orig.shape),
            full(W1a.shape),
            full(W1b.shape),
            full(b_c1.shape),
            full(W_c2.shape),
            full(b_c2.shape),
        ],
        out_specs=pl.BlockSpec((blk, d_out), lambda i: (i, 0)),
        out_shape=jax.ShapeDtypeStruct((n_pad, d_out), jnp.float32),
    )(agg2, deg2, W_ord, b_ord, W_gate, b_gate, W_orig, b_orig,
      W1a, W1b, b_c1, W_c2, b_c2)


def kernel(x, edge_index, W_ord, b_ord, W_gate, b_gate, W_orig, b_orig,
           W_c1, b_c1, W_c2, b_c2, gamma):
    n, d = x.shape
    e = edge_index.shape[1]
    n_pad = ((n + 1023) // 1024) * 1024

    epw = e // NW
    ch = next(c for c in (40, 32, 16, 8)
              if epw % (c * R_GRP) == 0 and (epw // (c * R_GRP)) % 2 == 0)
    ngrp = epw // (ch * R_GRP)

    idx3 = (edge_index.reshape(2, NW, ngrp, R_GRP, ch)
            .transpose(1, 2, 3, 0, 4)
            .reshape(NW * ngrp, 2 * R_GRP, ch))
    z2 = jnp.zeros((n_pad // NS, d), jnp.float32)
    z1 = jnp.zeros((n_pad // NS,), jnp.float32)

    agg2, deg2 = _sc_aggregate(x, idx3, z2, z1, n_pad, ch, ngrp)

    W1a = W_c1[:d]
    W1b = gamma * W_c1[d:]
    out = _tc_dense(
        agg2, deg2[:, :, None],
        W_ord, b_ord[:, None, :], W_gate, b_gate[:, None, :],
        W_orig, b_orig[None, :], W1a, W1b, b_c1[None, :],
        W_c2, b_c2[None, :], n_pad, blk=n_pad // 8)
    return out[:n]


# speedup vs baseline: 18.0313x; 1.0080x over previous
import jax
import jax.numpy as jnp
from jax import lax
from jax.experimental import pallas as pl
from jax.experimental.pallas import tpu as pltpu
from jax.experimental.pallas import tpu_sc as plsc

NC = 2
NS = 16
NW = NC * NS
R_GRP = 5


def _sc_aggregate(x, idx3, z2, z1, n_pad, ch, ngrp):
    n, d = x.shape
    rows_per_tile = n_pad // NS

    mesh = plsc.VectorSubcoreMesh(core_axis_name="c", subcore_axis_name="s")

    def body(x_hbm, idx_hbm, z2_hbm, z1_hbm, agg_out, deg_out,
             idxA, idxB, ones, agg_sh, deg_sh, *bufs):
        rows = bufs[0:R_GRP]
        gsem = bufs[R_GRP:2 * R_GRP]
        ssem = bufs[2 * R_GRP:3 * R_GRP]
        isemA = bufs[3 * R_GRP]
        isemB = bufs[3 * R_GRP + 1]
        dsem = bufs[3 * R_GRP + 2]
        c = lax.axis_index("c")
        s = lax.axis_index("s")
        wid = s * NC + c

        tile_lo = s * rows_per_tile
        pltpu.sync_copy(z2_hbm, agg_sh.at[pl.ds(tile_lo, rows_per_tile)])
        pltpu.sync_copy(z1_hbm, deg_sh.at[pl.ds(tile_lo, rows_per_tile)])
        for off in sorted({min(i * 16, ch - 16)
                           for i in range((ch + 15) // 16)}):
            ones[pl.ds(off, 16)] = jnp.ones((16,), jnp.float32)
        plsc.subcore_barrier()

        gbase = wid * ngrp

        def fire(idx, u):
            return pltpu.async_copy(x_hbm.at[idx.at[2 * u + 1]], rows[u],
                                    gsem[u])

        def drain_refill(idx_cur, idx_nxt, gds, refill):
            sds = []
            dds = []
            for u in range(R_GRP):
                gds[u].wait()
                sds.append(pltpu.async_copy(
                    rows[u], agg_sh.at[idx_cur.at[2 * u]], ssem[u],
                    add=True))
                dds.append(pltpu.async_copy(
                    ones, deg_sh.at[idx_cur.at[2 * u]], dsem, add=True))
            nds = []
            for u in range(R_GRP):
                sds[u].wait()
                if refill:
                    nds.append(fire(idx_nxt, u))
            for u in range(R_GRP):
                dds[u].wait()
            return nds

        pltpu.sync_copy(idx_hbm.at[gbase], idxA)
        g_in_flight = [fire(idxA, u) for u in range(R_GRP)]
        pltpu.async_copy(idx_hbm.at[gbase + 1], idxB, isemB)

        def pair(i, carry):
            pltpu.make_async_copy(idx_hbm.at[gbase], idxB, isemB).wait()
            gB = drain_refill(idxA, idxB, g_in_flight, True)
            pltpu.async_copy(idx_hbm.at[gbase + 2 * i + 2], idxA, isemA)
            pltpu.make_async_copy(idx_hbm.at[gbase], idxA, isemA).wait()
            gA = drain_refill(idxB, idxA, gB, True)
            for u in range(R_GRP):
                g_in_flight[u] = gA[u]
            pltpu.async_copy(idx_hbm.at[gbase + 2 * i + 3], idxB, isemB)
            return carry

        lax.fori_loop(0, ngrp // 2 - 1, pair, 0)

        pltpu.make_async_copy(idx_hbm.at[gbase], idxB, isemB).wait()
        gB = drain_refill(idxA, idxB, g_in_flight, True)
        drain_refill(idxB, idxB, gB, False)

        plsc.subcore_barrier()

        pltpu.sync_copy(agg_sh.at[pl.ds(tile_lo, rows_per_tile)],
                        agg_out.at[c, pl.ds(tile_lo, rows_per_tile)])
        pltpu.sync_copy(deg_sh.at[pl.ds(tile_lo, rows_per_tile)],
                        deg_out.at[c, pl.ds(tile_lo, rows_per_tile)])

    fn = pl.kernel(
        body,
        out_type=(jax.ShapeDtypeStruct((NC, n_pad, d), jnp.float32),
                  jax.ShapeDtypeStruct((NC, n_pad), jnp.float32)),
        mesh=mesh,
        scratch_types=[
            pltpu.VMEM((2 * R_GRP, ch), jnp.int32),
            pltpu.VMEM((2 * R_GRP, ch), jnp.int32),
            pltpu.VMEM((ch,), jnp.float32),
            pltpu.VMEM_SHARED((n_pad, d), jnp.float32),
            pltpu.VMEM_SHARED((n_pad,), jnp.float32),
        ] + [pltpu.VMEM((ch, d), jnp.float32) for _ in range(R_GRP)]
          + [pltpu.SemaphoreType.DMA for _ in range(2 * R_GRP + 3)],
    )
    return fn(x, idx3, z2, z1)


def _tc_dense_body(agg_ref, deg_ref, Word_ref, bord_ref, Wg_ref, bg_ref,
                   Worig_ref, borig_ref, W1a_ref, W1b_ref, b1_ref,
                   W2_ref, b2_ref, out_ref):
    a = agg_ref[0] + agg_ref[1]
    dg = deg_ref[0] + deg_ref[1]

    es = []
    ss = []
    for k in range(3):
        e = jnp.maximum(
            jnp.dot(a, Word_ref[k], preferred_element_type=jnp.float32)
            + dg * bord_ref[k], 0.0)
        s = (jnp.dot(e, Wg_ref[k], preferred_element_type=jnp.float32)
             + bg_ref[k])
        es.append(e)
        ss.append(s)

    m = jnp.maximum(jnp.maximum(ss[0], ss[1]), ss[2])
    p = [jnp.exp(s - m) for s in ss]
    tot = p[0] + p[1] + p[2]
    h_high = (p[0] * es[0] + p[1] * es[1] + p[2] * es[2]) / tot

    h_orig = jnp.maximum(
        jnp.dot(a, Worig_ref[...], preferred_element_type=jnp.float32)
        + dg * borig_ref[...], 0.0)

    z1 = jnp.maximum(
        jnp.dot(h_orig, W1a_ref[...], preferred_element_type=jnp.float32)
        + jnp.dot(h_high, W1b_ref[...], preferred_element_type=jnp.float32)
        + b1_ref[...], 0.0)
    out_ref[...] = (jnp.dot(z1, W2_ref[...], preferred_element_type=jnp.float32)
                    + b2_ref[...])


def _tc_dense(agg2, deg2, W_ord, b_ord, W_gate, b_gate, W_orig, b_orig,
              W1a, W1b, b_c1, W_c2, b_c2, n_pad, blk):
    d = agg2.shape[-1]
    d_out = W_c2.shape[-1]
    grid = n_pad // blk

    def full(shape):
        return pl.BlockSpec(shape, lambda i, _s=shape: tuple(0 for _ in _s))

    return pl.pallas_call(
        _tc_dense_body,
        grid=(grid,),
        in_specs=[
            pl.BlockSpec((NC, blk, d), lambda i: (0, i, 0)),
            pl.BlockSpec((NC, blk, 1), lambda i: (0, i, 0)),
            full(W_ord.shape),
            full(b_ord.shape),
            full(W_gate.shape),
            full(b_gate.shape),
            full(W_orig.shape),
            full(b_orig.shape),
            full(W1a.shape),
            full(W1b.shape),
            full(b_c1.shape),
            full(W_c2.shape),
            full(b_c2.shape),
        ],
        out_specs=pl.BlockSpec((blk, d_out), lambda i: (i, 0)),
        out_shape=jax.ShapeDtypeStruct((n_pad, d_out), jnp.float32),
    )(agg2, deg2, W_ord, b_ord, W_gate, b_gate, W_orig, b_orig,
      W1a, W1b, b_c1, W_c2, b_c2)


def kernel(x, edge_index, W_ord, b_ord, W_gate, b_gate, W_orig, b_orig,
           W_c1, b_c1, W_c2, b_c2, gamma):
    n, d = x.shape
    e = edge_index.shape[1]
    n_pad = ((n + 1023) // 1024) * 1024

    epw = e // NW
    ch = next(c for c in (40, 32, 16, 8)
              if epw % (c * R_GRP) == 0 and (epw // (c * R_GRP)) % 2 == 0)
    ngrp = epw // (ch * R_GRP)

    idx3 = (edge_index.reshape(2, NW, ngrp, R_GRP, ch)
            .transpose(1, 2, 3, 0, 4)
            .reshape(NW * ngrp, 2 * R_GRP, ch))
    z2 = jnp.zeros((n_pad // NS, d), jnp.float32)
    z1 = jnp.zeros((n_pad // NS,), jnp.float32)

    agg2, deg2 = _sc_aggregate(x, idx3, z2, z1, n_pad, ch, ngrp)

    W1a = W_c1[:d]
    W1b = gamma * W_c1[d:]
    out = _tc_dense(
        agg2, deg2[:, :, None],
        W_ord, b_ord[:, None, :], W_gate, b_gate[:, None, :],
        W_orig, b_orig[None, :], W1a, W1b, b_c1[None, :],
        W_c2, b_c2[None, :], n_pad, blk=n_pad // 4)
    return out[:n]
